# baseline (speedup 1.0000x reference)
import jax
import jax.numpy as jnp
from jax import lax
from jax.experimental import pallas as pl
from jax.experimental.pallas import tpu as pltpu
from jax.experimental.pallas import tpu_sc as plsc

N_ROWS = 4096
N_FIELDS = 26
N_CLASSES = 1000
ROW_W = N_FIELDS * N_CLASSES
NC = 2
NS = 16
NW = NC * NS
ROWS_PER_W = N_ROWS // NW
R = 4
N_ROUNDS = ROWS_PER_W // R


F_PAD = 32


def _scatter_row(x_v, buf_v, row_local, buf_base, value):
    for h in range(2):
        jvec = lax.iota(jnp.int32, 16) + h * 16
        valid = jvec < N_FIELDS
        xv = x_v[pl.ds(row_local * F_PAD + h * 16, 16)]
        idx = jnp.where(xv == 0, 0, xv - 1)
        pos = buf_base + jvec * N_CLASSES + idx
        pos = jnp.where(valid, pos, buf_base)
        plsc.store_scatter(buf_v, [pos],
                           jnp.full((16,), value, jnp.float32), mask=valid)


def _sc_body(x_hbm, out_hbm, x_v, buf_v):
    c = lax.axis_index("c")
    s = lax.axis_index("s")
    wid = s * NC + c
    base = wid * ROWS_PER_W

    pltpu.sync_copy(x_hbm.at[pl.ds(base * F_PAD, ROWS_PER_W * F_PAD)],
                    x_v)

    def _zero(i, carry):
        buf_v[pl.ds(i * 16, 16)] = jnp.zeros((16,), jnp.float32)
        return carry
    lax.fori_loop(0, R * ROW_W // 16, _zero, 0)

    def _round(g, carry):
        row0 = g * R
        for rr in range(R):
            _scatter_row(x_v, buf_v, row0 + rr, rr * ROW_W, -1.0)
        for rr in range(R):
            pltpu.sync_copy(buf_v.at[pl.ds(rr * ROW_W, ROW_W)],
                            out_hbm.at[base + row0 + rr])
        for rr in range(R):
            _scatter_row(x_v, buf_v, row0 + rr, rr * ROW_W, 0.0)
        return carry
    lax.fori_loop(0, N_ROUNDS, _round, 0)


def kernel(x):
    mesh = plsc.VectorSubcoreMesh(core_axis_name="c", subcore_axis_name="s")
    kfn = pl.kernel(
        _sc_body,
        mesh=mesh,
        out_type=jax.ShapeDtypeStruct((N_ROWS, ROW_W), jnp.float32),
        scratch_types=[
            pltpu.VMEM((ROWS_PER_W * F_PAD,), jnp.int32),
            pltpu.VMEM((R * ROW_W,), jnp.float32),
        ],
        compiler_params=pltpu.CompilerParams(
            needs_layout_passes=False, use_tc_tiling_on_sc=False),
    )
    xp = jnp.pad(x, ((0, 0), (0, F_PAD - N_FIELDS)))
    return kfn(xp.reshape(-1))


# speedup vs baseline: 8.4473x; 8.4473x over previous
import jax
import jax.numpy as jnp
from jax import lax
from jax.experimental import pallas as pl

N_ROWS = 4096
N_FIELDS = 26
N_CLASSES = 1000


def _onehot_t_kernel(xt_ref, y_ref):
    xj = xt_ref[...].reshape(1, N_ROWS)
    idx = jnp.where(xj == 0, 0, xj - 1)
    k = lax.broadcasted_iota(jnp.int32, (N_CLASSES, N_ROWS), 0)
    y_ref[...] = jnp.where(k == idx, -1.0, 0.0)


def kernel(x):
    xt = x.T.reshape(N_FIELDS, 1, N_ROWS)
    y = pl.pallas_call(
        _onehot_t_kernel,
        grid=(N_FIELDS,),
        in_specs=[pl.BlockSpec((1, 1, N_ROWS), lambda j: (j, 0, 0))],
        out_specs=pl.BlockSpec((N_CLASSES, N_ROWS), lambda j: (j, 0)),
        out_shape=jax.ShapeDtypeStruct((N_FIELDS * N_CLASSES, N_ROWS),
                                       jnp.float32),
    )(xt)
    return y.T
